# baseline (speedup 1.0000x reference)
import jax
import jax.numpy as jnp
from jax import lax
from jax.experimental import pallas as pl
from jax.experimental.pallas import tpu as pltpu
from jax.experimental.pallas import tpu_sc as plsc

NS = 16
L = 16
B = 16384
BT = B // NS
V = 5000
D = 64
WT = 8
FPT = D // WT
CRR = 48
NCH = V // L
NTL = V - NCH * L
SCALE = 1.0 / B


def _sc_body(idx_hbm, tableT_hbm, out_hbm, idx_v, cnt_v, cntm_v, tbl_v,
             acc_v, rows_v, fin_v, shcnt_v, shacc_v, semt, semi):
    sid = lax.axis_index("s")
    fgrp = lax.bitwise_and(sid, WT - 1)
    half = lax.shift_right_logical(sid, 3)
    scope = jax.named_scope

    start = pl.multiple_of(sid * BT, BT)
    pltpu.async_copy(idx_hbm.at[pl.ds(start, BT)], idx_v, semi)
    fstart = pl.multiple_of(fgrp * FPT, FPT)
    pltpu.async_copy(tableT_hbm.at[pl.ds(fstart, FPT)], tbl_v, semt)

    def zbody(i, _):
        for j in range(8):
            cnt_v[i, pl.ds(j * L, L)] = jnp.zeros((L,), jnp.float32)
        return 0

    with scope("zero_cnt"):
        lax.fori_loop(0, CRR, zbody, 0)
        iota = lax.iota(jnp.int32, L)
        for k in range(CRR // L):
            rows_v[pl.ds(k * L, L)] = iota + k * L

    @pl.when(sid == 0)
    def _():
        with scope("init_shcnt"):
            pltpu.sync_copy(cnt_v, shcnt_v)

    ones = jnp.ones((L,), jnp.float32)
    with scope("idx_wait"):
        pltpu.make_async_copy(
            idx_hbm.at[pl.ds(start, BT)], idx_v, semi).wait()

    def hbody(i, _):
        iv = idx_v[pl.ds(i * L, L)]
        r = lax.shift_right_logical(iv, 7)
        c = lax.bitwise_and(iv, 127)
        plsc.addupdate_scatter(cnt_v, [r, c], ones)
        return 0

    with scope("hist"):
        lax.fori_loop(0, BT // L, hbody, 0)
    with scope("barrier0"):
        plsc.subcore_barrier()
    with scope("merge_add"):
        pltpu.sync_copy(cnt_v, shcnt_v.at[rows_v], add=True)
    with scope("barrier1"):
        plsc.subcore_barrier()

    with scope("cnt_fetch"):
        pltpu.sync_copy(shcnt_v, cntm_v)
    with scope("tbl_wait"):
        pltpu.make_async_copy(
            tableT_hbm.at[pl.ds(fstart, FPT)], tbl_v, semt).wait()

    def gbody(gi, a):
        cv = cntm_v[lax.shift_right_logical(gi, 3),
                    pl.ds(lax.bitwise_and(gi, 7) * L, L)]
        col = gi * L
        return tuple(
            a[f] + cv * tbl_v[f, pl.ds(col, L)] for f in range(FPT))

    with scope("wsum"):
        acc0 = tuple(jnp.zeros((L,), jnp.float32) for _ in range(FPT))
        g0 = half * (NCH // 2)
        acc = lax.fori_loop(g0, g0 + NCH // 2, gbody, acc0)
        lane = lax.iota(jnp.int32, L)
        cv = cntm_v[NCH // 8, pl.ds((NCH % 8) * L, L)]
        cvt = jnp.where((lane < NTL) & (half == 1), cv, 0.0)
        tcol = NCH * L + (lane & (NTL - 1))
        acc = tuple(
            a + cvt * plsc.load_gather(
                tbl_v, [jnp.full((L,), f, jnp.int32), tcol])
            for f, a in enumerate(acc))
        res = jnp.zeros((L,), jnp.float32)
        for f in range(FPT):
            tot = lax.broadcast(jnp.sum(acc[f], axis=0), (L,))
            res = jnp.where(lane == f, tot, res)
        acc_v[pl.ds(0, L)] = res
        off = pl.multiple_of(half * D + fgrp * FPT, FPT)
        pltpu.sync_copy(acc_v.at[pl.ds(0, FPT)],
                        shacc_v.at[pl.ds(off, FPT)])

    with scope("finalize"):
        plsc.subcore_barrier()

        @pl.when(sid == 0)
        def _():
            pltpu.sync_copy(shacc_v, fin_v)
            for k in range(D // L):
                s = (fin_v[pl.ds(k * L, L)]
                     + fin_v[pl.ds(D + k * L, L)])
                acc_v[pl.ds(k * L, L)] = s * SCALE
            pltpu.sync_copy(acc_v, out_hbm)


def kernel(idxs, table):
    mesh = plsc.VectorSubcoreMesh(
        core_axis_name="c", subcore_axis_name="s", num_cores=1)
    f = pl.kernel(
        _sc_body,
        out_type=jax.ShapeDtypeStruct((D,), jnp.float32),
        mesh=mesh,
        scratch_types=[
            pltpu.VMEM((BT,), jnp.int32),
            pltpu.VMEM((CRR, 128), jnp.float32),
            pltpu.VMEM((CRR, 128), jnp.float32),
            pltpu.VMEM((FPT, V), jnp.float32),
            pltpu.VMEM((D,), jnp.float32),
            pltpu.VMEM((CRR,), jnp.int32),
            pltpu.VMEM((2 * D,), jnp.float32),
            pltpu.VMEM_SHARED((CRR, 128), jnp.float32),
            pltpu.VMEM_SHARED((2 * D,), jnp.float32),
            pltpu.SemaphoreType.DMA,
            pltpu.SemaphoreType.DMA,
        ],
        compiler_params=pltpu.CompilerParams(needs_layout_passes=False),
    )
    return f(idxs, table.T)


# speedup vs baseline: 1.0020x; 1.0020x over previous
import jax
import jax.numpy as jnp
from jax import lax
from jax.experimental import pallas as pl
from jax.experimental.pallas import tpu as pltpu
from jax.experimental.pallas import tpu_sc as plsc

NS = 16
L = 16
B = 16384
BT = B // NS
V = 5000
D = 64
WT = 8
FPT = D // WT
CRR = 48
NCH = V // L
NTL = V - NCH * L
SCALE = 1.0 / B


def _sc_body(idx_hbm, tableT_hbm, out_hbm, idx_v, cnt_v, cntm_v, tbl_v,
             acc_v, rows_v, fin_v, shcnt_v, shacc_v, semt, semi):
    sid = lax.axis_index("s")
    fgrp = lax.bitwise_and(sid, WT - 1)
    half = lax.shift_right_logical(sid, 3)
    scope = jax.named_scope

    start = pl.multiple_of(sid * BT, BT)
    pltpu.async_copy(idx_hbm.at[pl.ds(start, BT)], idx_v, semi)
    fstart = pl.multiple_of(fgrp * FPT, FPT)
    pltpu.async_copy(tableT_hbm.at[pl.ds(fstart, FPT)], tbl_v, semt)

    def zbody(i, _):
        for j in range(8):
            cnt_v[i, pl.ds(j * L, L)] = jnp.zeros((L,), jnp.float32)
        return 0

    with scope("zero_cnt"):
        lax.fori_loop(0, CRR, zbody, 0)
        iota = lax.iota(jnp.int32, L)
        for k in range(CRR // L):
            rows_v[pl.ds(k * L, L)] = iota + k * L

    @pl.when(sid == 0)
    def _():
        with scope("init_shcnt"):
            pltpu.sync_copy(cnt_v, shcnt_v)

    ones = jnp.ones((L,), jnp.float32)
    with scope("idx_wait"):
        pltpu.make_async_copy(
            idx_hbm.at[pl.ds(start, BT)], idx_v, semi).wait()

    def hbody(i, _):
        iv = idx_v[pl.ds(i * L, L)]
        r = lax.shift_right_logical(iv, 7)
        c = lax.bitwise_and(iv, 127)
        plsc.addupdate_scatter(cnt_v, [r, c], ones)
        return 0

    with scope("hist"):
        lax.fori_loop(0, BT // L, hbody, 0)
    with scope("barrier0"):
        plsc.subcore_barrier()
    with scope("merge_add"):
        pltpu.sync_copy(cnt_v, shcnt_v.at[rows_v], add=True)
    with scope("barrier1"):
        plsc.subcore_barrier()

    with scope("cnt_fetch"):
        pltpu.async_copy(shcnt_v, cntm_v, semi)
    with scope("tbl_wait"):
        pltpu.make_async_copy(
            tableT_hbm.at[pl.ds(fstart, FPT)], tbl_v, semt).wait()
    with scope("cnt_wait"):
        pltpu.make_async_copy(shcnt_v, cntm_v, semi).wait()

    def gbody(gi, a):
        cv = cntm_v[lax.shift_right_logical(gi, 3),
                    pl.ds(lax.bitwise_and(gi, 7) * L, L)]
        col = gi * L
        return tuple(
            a[f] + cv * tbl_v[f, pl.ds(col, L)] for f in range(FPT))

    with scope("wsum"):
        acc0 = tuple(jnp.zeros((L,), jnp.float32) for _ in range(FPT))
        g0 = half * (NCH // 2)
        acc = lax.fori_loop(g0, g0 + NCH // 2, gbody, acc0)
        lane = lax.iota(jnp.int32, L)
        cv = cntm_v[NCH // 8, pl.ds((NCH % 8) * L, L)]
        cvt = jnp.where((lane < NTL) & (half == 1), cv, 0.0)
        tcol = NCH * L + (lane & (NTL - 1))
        acc = tuple(
            a + cvt * plsc.load_gather(
                tbl_v, [jnp.full((L,), f, jnp.int32), tcol])
            for f, a in enumerate(acc))
        res = jnp.zeros((L,), jnp.float32)
        for f in range(FPT):
            tot = lax.broadcast(jnp.sum(acc[f], axis=0), (L,))
            res = jnp.where(lane == f, tot, res)
        acc_v[pl.ds(0, L)] = res
        off = pl.multiple_of(half * D + fgrp * FPT, FPT)
        pltpu.sync_copy(acc_v.at[pl.ds(0, FPT)],
                        shacc_v.at[pl.ds(off, FPT)])

    with scope("finalize"):
        plsc.subcore_barrier()

        @pl.when(sid == 0)
        def _():
            pltpu.sync_copy(shacc_v, fin_v)
            for k in range(D // L):
                s = (fin_v[pl.ds(k * L, L)]
                     + fin_v[pl.ds(D + k * L, L)])
                acc_v[pl.ds(k * L, L)] = s * SCALE
            pltpu.sync_copy(acc_v, out_hbm)


def kernel(idxs, table):
    mesh = plsc.VectorSubcoreMesh(
        core_axis_name="c", subcore_axis_name="s", num_cores=1)
    f = pl.kernel(
        _sc_body,
        out_type=jax.ShapeDtypeStruct((D,), jnp.float32),
        mesh=mesh,
        scratch_types=[
            pltpu.VMEM((BT,), jnp.int32),
            pltpu.VMEM((CRR, 128), jnp.float32),
            pltpu.VMEM((CRR, 128), jnp.float32),
            pltpu.VMEM((FPT, V), jnp.float32),
            pltpu.VMEM((D,), jnp.float32),
            pltpu.VMEM((CRR,), jnp.int32),
            pltpu.VMEM((2 * D,), jnp.float32),
            pltpu.VMEM_SHARED((CRR, 128), jnp.float32),
            pltpu.VMEM_SHARED((2 * D,), jnp.float32),
            pltpu.SemaphoreType.DMA,
            pltpu.SemaphoreType.DMA,
        ],
        compiler_params=pltpu.CompilerParams(needs_layout_passes=False),
    )
    return f(idxs, table.T)


# speedup vs baseline: 1.0042x; 1.0023x over previous
import jax
import jax.numpy as jnp
from jax import lax
from jax.experimental import pallas as pl
from jax.experimental.pallas import tpu as pltpu
from jax.experimental.pallas import tpu_sc as plsc

NS = 16
L = 16
B = 16384
BT = B // NS
V = 5000
D = 64
NG = 8
FPT = D // NG
CRR = 48
NCH = V // L
NTL = V - NCH * L
SCALE = 1.0 / B


def _sc_body(idx_hbm, tableT_hbm, out_hbm, idx_v, cnt_v, cntm_v, tbl_v,
             acc_v, rows_v, fin_v, shcnt_v, shacc_v, semt, semi):
    sid = lax.axis_index("s")
    fgrp = lax.bitwise_and(sid, NG - 1)
    half = lax.shift_right_logical(sid, 3)

    start = pl.multiple_of(sid * BT, BT)
    pltpu.async_copy(idx_hbm.at[pl.ds(start, BT)], idx_v, semi)
    fstart = pl.multiple_of(fgrp * FPT, FPT)
    pltpu.async_copy(tableT_hbm.at[pl.ds(fstart, FPT)], tbl_v, semt)

    def zbody(i, _):
        for j in range(8):
            cnt_v[i, pl.ds(j * L, L)] = jnp.zeros((L,), jnp.float32)
        return 0

    lax.fori_loop(0, CRR, zbody, 0)
    iota = lax.iota(jnp.int32, L)
    for k in range(CRR // L):
        rows_v[pl.ds(k * L, L)] = iota + k * L

    @pl.when(sid == 0)
    def _():
        pltpu.sync_copy(cnt_v, shcnt_v)

    ones = jnp.ones((L,), jnp.float32)
    pltpu.make_async_copy(idx_hbm.at[pl.ds(start, BT)], idx_v, semi).wait()

    def hbody(i, _):
        iv = idx_v[pl.ds(i * L, L)]
        r = lax.shift_right_logical(iv, 7)
        c = lax.bitwise_and(iv, 127)
        plsc.addupdate_scatter(cnt_v, [r, c], ones)
        return 0

    lax.fori_loop(0, BT // L, hbody, 0)
    plsc.subcore_barrier()
    pltpu.sync_copy(cnt_v, shcnt_v.at[rows_v], add=True)
    plsc.subcore_barrier()

    pltpu.async_copy(shcnt_v, cntm_v, semi)
    pltpu.make_async_copy(
        tableT_hbm.at[pl.ds(fstart, FPT)], tbl_v, semt).wait()
    pltpu.make_async_copy(shcnt_v, cntm_v, semi).wait()

    def gbody(gi, a):
        cv = cntm_v[lax.shift_right_logical(gi, 3),
                    pl.ds(lax.bitwise_and(gi, 7) * L, L)]
        col = gi * L
        return tuple(
            a[f] + cv * tbl_v[f, pl.ds(col, L)] for f in range(FPT))

    acc0 = tuple(jnp.zeros((L,), jnp.float32) for _ in range(FPT))
    g0 = half * (NCH // 2)
    acc = lax.fori_loop(g0, g0 + NCH // 2, gbody, acc0)
    lane = lax.iota(jnp.int32, L)
    cv = cntm_v[NCH // 8, pl.ds((NCH % 8) * L, L)]
    cvt = jnp.where((lane < NTL) & (half == 1), cv, 0.0)
    tcol = NCH * L + (lane & (NTL - 1))
    acc = tuple(
        a + cvt * plsc.load_gather(
            tbl_v, [jnp.full((L,), f, jnp.int32), tcol])
        for f, a in enumerate(acc))
    res = jnp.zeros((L,), jnp.float32)
    for f in range(FPT):
        tot = lax.broadcast(jnp.sum(acc[f], axis=0), (L,))
        res = jnp.where(lane == f, tot, res)
    acc_v[pl.ds(0, L)] = res
    off = pl.multiple_of(half * D + fgrp * FPT, FPT)
    pltpu.sync_copy(acc_v.at[pl.ds(0, FPT)], shacc_v.at[pl.ds(off, FPT)])

    plsc.subcore_barrier()

    @pl.when(sid == 0)
    def _():
        pltpu.sync_copy(shacc_v, fin_v)
        for k in range(D // L):
            s = fin_v[pl.ds(k * L, L)] + fin_v[pl.ds(D + k * L, L)]
            acc_v[pl.ds(k * L, L)] = s * SCALE
        pltpu.sync_copy(acc_v, out_hbm)


def kernel(idxs, table):
    mesh = plsc.VectorSubcoreMesh(
        core_axis_name="c", subcore_axis_name="s", num_cores=1)
    f = pl.kernel(
        _sc_body,
        out_type=jax.ShapeDtypeStruct((D,), jnp.float32),
        mesh=mesh,
        scratch_types=[
            pltpu.VMEM((BT,), jnp.int32),
            pltpu.VMEM((CRR, 128), jnp.float32),
            pltpu.VMEM((CRR, 128), jnp.float32),
            pltpu.VMEM((FPT, V), jnp.float32),
            pltpu.VMEM((D,), jnp.float32),
            pltpu.VMEM((CRR,), jnp.int32),
            pltpu.VMEM((2 * D,), jnp.float32),
            pltpu.VMEM_SHARED((CRR, 128), jnp.float32),
            pltpu.VMEM_SHARED((2 * D,), jnp.float32),
            pltpu.SemaphoreType.DMA,
            pltpu.SemaphoreType.DMA,
        ],
        compiler_params=pltpu.CompilerParams(needs_layout_passes=False),
    )
    return f(idxs, table.T)
